# baseline (speedup 1.0000x reference)
import functools

import jax
import jax.numpy as jnp
from jax import lax
from jax.experimental import pallas as pl
from jax.experimental.pallas import tpu as pltpu
from jax.experimental.pallas import tpu_sc as plsc

_DIM = 64
_ENT = 1000000
_BATCH = 16384
_MARGIN = 6.0

_NC = 2
_NS = 16
_LANES = 16
_DPC = _DIM // _NC
_BPT = _BATCH // _NS
_K = _BPT // 128


def _sc_partial_kernel(ent_t, rel_t,
                       ph_hbm, pt_hbm, pr_hbm,
                       nh_hbm, nt_hbm, nr_hbm,
                       out_hbm,
                       iph, ipt, ipr, inh, int_, inr,
                       ghp, gtp, grp, ghn, gtn, grn,
                       accp, accn,
                       row_sh,
                       sem_e, sem_r, sem_g):
    c = lax.axis_index("c")
    sid = lax.axis_index("s")
    d0 = c * _DPC
    t8 = sid * _K

    pltpu.sync_copy(ph_hbm.at[pl.ds(t8, _K)], iph)
    pltpu.sync_copy(pt_hbm.at[pl.ds(t8, _K)], ipt)
    pltpu.sync_copy(pr_hbm.at[pl.ds(t8, _K)], ipr)
    pltpu.sync_copy(nh_hbm.at[pl.ds(t8, _K)], inh)
    pltpu.sync_copy(nt_hbm.at[pl.ds(t8, _K)], int_)
    pltpu.sync_copy(nr_hbm.at[pl.ds(t8, _K)], inr)

    def zero(b, _):
        bs = pl.ds(b * _LANES, _LANES)
        z = jnp.zeros((_LANES,), jnp.float32)
        accp[bs] = z
        accn[bs] = z
        return 0
    lax.fori_loop(0, _BPT // _LANES, zero, 0)

    @pl.when(sid == 0)
    def _():
        pltpu.async_copy(ent_t.at[d0], row_sh, sem_e).wait()

    plsc.subcore_barrier()

    def dim_step(j, _):
        dim = d0 + j
        dim_next = jnp.minimum(dim + 1, _DIM - 1)

        cps = []
        for idx, dst in ((iph, ghp), (ipt, gtp), (inh, ghn), (int_, gtn)):
            for kk in range(_K):
                cps.append(pltpu.async_copy(
                    row_sh.at[idx.at[kk]], dst.at[pl.ds(kk * 128, 128)],
                    sem_g))
        for cp in cps:
            cp.wait()

        plsc.subcore_barrier()

        @pl.when(sid == 1)
        def _():
            pltpu.async_copy(rel_t.at[dim], row_sh, sem_r).wait()

        plsc.subcore_barrier()

        cps = []
        for idx, dst in ((ipr, grp), (inr, grn)):
            for kk in range(_K):
                cps.append(pltpu.async_copy(
                    row_sh.at[idx.at[kk]], dst.at[pl.ds(kk * 128, 128)],
                    sem_g))
        for cp in cps:
            cp.wait()

        plsc.subcore_barrier()

        @pl.when(sid == 0)
        def _():
            pltpu.async_copy(ent_t.at[dim_next], row_sh, sem_e)

        def acc_step(b, _):
            bs = pl.ds(b * _LANES, _LANES)
            dp = ghp[bs] + grp[bs] - gtp[bs]
            accp[bs] = accp[bs] + dp * dp
            dn = ghn[bs] + grn[bs] - gtn[bs]
            accn[bs] = accn[bs] + dn * dn
            return 0
        lax.fori_loop(0, _BPT // _LANES, acc_step, 0)

        @pl.when(sid == 0)
        def _():
            pltpu.make_async_copy(ent_t.at[dim_next], row_sh, sem_e).wait()

        plsc.subcore_barrier()

        return 0

    lax.fori_loop(0, _DPC, dim_step, 0)

    base = c * (2 * _BATCH) + sid * _BPT
    pltpu.sync_copy(accp, out_hbm.at[pl.ds(base, _BPT)])
    pltpu.sync_copy(accn, out_hbm.at[pl.ds(base + _BATCH, _BPT)])


_sc_partial = functools.partial(
    pl.kernel,
    mesh=plsc.VectorSubcoreMesh(core_axis_name="c", subcore_axis_name="s"),
    compiler_params=pltpu.CompilerParams(needs_layout_passes=False),
    out_type=jax.ShapeDtypeStruct((2 * 2 * _BATCH,), jnp.float32),
    scratch_types=[
        pltpu.VMEM((_K, 128), jnp.int32),
        pltpu.VMEM((_K, 128), jnp.int32),
        pltpu.VMEM((_K, 128), jnp.int32),
        pltpu.VMEM((_K, 128), jnp.int32),
        pltpu.VMEM((_K, 128), jnp.int32),
        pltpu.VMEM((_K, 128), jnp.int32),
        pltpu.VMEM((_BPT,), jnp.float32),
        pltpu.VMEM((_BPT,), jnp.float32),
        pltpu.VMEM((_BPT,), jnp.float32),
        pltpu.VMEM((_BPT,), jnp.float32),
        pltpu.VMEM((_BPT,), jnp.float32),
        pltpu.VMEM((_BPT,), jnp.float32),
        pltpu.VMEM((_BPT,), jnp.float32),
        pltpu.VMEM((_BPT,), jnp.float32),
        pltpu.VMEM_SHARED((_ENT,), jnp.float32),
        pltpu.SemaphoreType.DMA,
        pltpu.SemaphoreType.DMA,
        pltpu.SemaphoreType.DMA,
    ],
)(_sc_partial_kernel)


def _epilogue_kernel(parts_ref, out_ref):
    pos = parts_ref[0] + parts_ref[2]
    neg = parts_ref[1] + parts_ref[3]
    x = jnp.maximum(jnp.sqrt(pos) - jnp.sqrt(neg), -_MARGIN)
    out_ref[...] = (jnp.sum(x) / _BATCH + _MARGIN).reshape(1, 1)


def kernel(batch_corrects, batch_corrupts, ent_emb, rel_emb):
    ent_t = ent_emb.T
    rel_t = rel_emb.T

    ph = batch_corrects[:, 0].reshape(128, 128)
    pt = batch_corrects[:, 1].reshape(128, 128)
    pr = batch_corrects[:, 2].reshape(128, 128)
    nh = batch_corrupts[:, 0].reshape(128, 128)
    nt = batch_corrupts[:, 1].reshape(128, 128)
    nr = batch_corrupts[:, 2].reshape(128, 128)

    parts = _sc_partial(ent_t, rel_t, ph, pt, pr, nh, nt, nr)

    loss = pl.pallas_call(
        _epilogue_kernel,
        out_shape=jax.ShapeDtypeStruct((1, 1), jnp.float32),
    )(parts.reshape(4, 128, 128))
    return loss.reshape(1)
